# baseline (speedup 1.0000x reference)
import functools

import jax
import jax.numpy as jnp
from jax import lax
from jax.experimental import pallas as pl
from jax.experimental.pallas import tpu as pltpu
from jax.experimental.pallas import tpu_sc as plsc

N = 10000
NP = 10240
E = 160000
F = 256
NS = 16
EPT = E // NS
EPP = 10240
NH = NP // 2
ACC = NH + 8
CH = 64
HLF = EPP // 2
HCH = HLF // CH
SPT = NH // NS
RB = 1024
GRID = NP // RB

_mesh = plsc.VectorSubcoreMesh(core_axis_name="c", subcore_axis_name="s")
_sc_params = pltpu.CompilerParams(needs_layout_passes=False)


@functools.partial(
    pl.kernel,
    out_type=jax.ShapeDtypeStruct((2, NP), jnp.float32),
    mesh=_mesh,
    scratch_types=[
        pltpu.VMEM((EPT,), jnp.int32),
        pltpu.VMEM((NP,), jnp.float32),
        pltpu.VMEM((16,), jnp.int32),
        pltpu.VMEM((NP // NS,), jnp.float32),
        pltpu.VMEM((NP // NS,), jnp.float32),
        pltpu.VMEM_SHARED((NS * NP,), jnp.float32),
    ],
    compiler_params=_sc_params,
)
def _deg_kernel(graph_hbm, out_hbm, idx_v, hist_v, sbuf, tmp_v, res_v, shbuf):
    c = lax.axis_index("c")
    s = lax.axis_index("s")
    seg = NP // NS
    pltpu.sync_copy(graph_hbm.at[c, s], idx_v)

    def _zero(r, carry):
        hist_v[pl.ds(r * 16, 16)] = jnp.zeros((16,), jnp.float32)
        return carry

    lax.fori_loop(0, NP // 16, _zero, 0)

    lanes = lax.iota(jnp.int32, 16)

    def _hist(i, carry):
        key = lax.sort(idx_v[pl.ds(i * 16, 16)])
        sbuf[...] = key
        prv = plsc.load_gather(sbuf, [jnp.maximum(lanes - 1, 0)])
        nxt = plsc.load_gather(sbuf, [jnp.minimum(lanes + 1, 15)])
        is_start = (lanes == 0) | (key != prv)
        is_end = (lanes == 15) | (key != nxt)
        start = plsc.cummax(jnp.where(is_start, lanes, -1))
        cnt = (lanes - start + 1).astype(jnp.float32)
        plsc.addupdate_scatter(hist_v, [key], cnt, mask=is_end)
        return carry

    lax.fori_loop(0, EPT // 16, _hist, 0)

    pltpu.sync_copy(hist_v, shbuf.at[pl.ds(s * NP, NP)])
    plsc.subcore_barrier()

    def _rzero(r, carry):
        res_v[pl.ds(r * 16, 16)] = jnp.zeros((16,), jnp.float32)
        return carry

    lax.fori_loop(0, seg // 16, _rzero, 0)
    for t in range(NS):
        pltpu.sync_copy(shbuf.at[pl.ds(t * NP + s * seg, seg)], tmp_v)

        def _radd(r, carry):
            sl = pl.ds(r * 16, 16)
            res_v[sl] = res_v[sl] + tmp_v[sl]
            return carry

        lax.fori_loop(0, seg // 16, _radd, 0)
    pltpu.sync_copy(res_v, out_hbm.at[c, pl.ds(s * seg, seg)])


@functools.partial(
    pl.kernel,
    out_type=[
        jax.ShapeDtypeStruct((2, NS, EPP), jnp.int32),
        jax.ShapeDtypeStruct((2, NS, EPP), jnp.int32),
        jax.ShapeDtypeStruct((2, NS, 16), jnp.int32),
    ],
    mesh=_mesh,
    scratch_types=[
        pltpu.VMEM((EPT,), jnp.int32),
        pltpu.VMEM((EPT,), jnp.int32),
        pltpu.VMEM((EPP,), jnp.int32),
        pltpu.VMEM((EPP,), jnp.int32),
        pltpu.VMEM((16,), jnp.int32),
    ],
    compiler_params=_sc_params,
)
def _part_kernel(graph_hbm, src_out, dst_out, cnt_out,
                 src_v, dst_v, csrc_v, cdst_v, cnt_v):
    c = lax.axis_index("c")
    s = lax.axis_index("s")
    pltpu.sync_copy(graph_hbm.at[0, s], src_v)
    pltpu.sync_copy(graph_hbm.at[1, s], dst_v)

    lo = c * NH

    zeros = jnp.zeros((16,), jnp.int32)
    dumv = jnp.full((16,), NH, jnp.int32)

    def _fill(r, carry):
        sl = pl.ds(r * 16, 16)
        csrc_v[sl] = zeros
        cdst_v[sl] = dumv
        return carry

    lax.fori_loop(0, EPP // 16, _fill, 0)

    def _compact(i, off):
        sl = pl.ds(i * 16, 16)
        srcs = src_v[sl]
        dsts = dst_v[sl]
        mask = (dsts >= lo) & (dsts < lo + NH)
        cs = plsc.cumsum(mask.astype(jnp.int32))
        pos = jnp.where(mask, off + cs - 1, 0)
        plsc.store_scatter(csrc_v, [pos], srcs, mask=mask)
        plsc.store_scatter(cdst_v, [pos], dsts - lo, mask=mask)
        return off + jnp.max(cs)

    cnt = lax.fori_loop(0, EPT // 16, _compact, jnp.int32(0))

    cnt_v[...] = jnp.full((16,), 1, jnp.int32) * cnt
    pltpu.sync_copy(csrc_v, src_out.at[c, s])
    pltpu.sync_copy(cdst_v, dst_out.at[c, s])
    pltpu.sync_copy(cnt_v, cnt_out.at[c, s])


@functools.partial(
    pl.kernel,
    out_type=jax.ShapeDtypeStruct((2, NH, 2, F // 2), jnp.float32),
    mesh=_mesh,
    scratch_types=[
        pltpu.VMEM((HLF,), jnp.int32),
        pltpu.VMEM((HCH, CH), jnp.int32),
        pltpu.VMEM((16,), jnp.int32),
        pltpu.VMEM((CH, 2, F // 2), jnp.float32),
        pltpu.VMEM((CH, 2, F // 2), jnp.float32),
        pltpu.VMEM_SHARED((ACC, 2, F // 2), jnp.float32),
        pltpu.SemaphoreType.DMA,
        pltpu.SemaphoreType.DMA,
        pltpu.SemaphoreType.DMA,
        pltpu.SemaphoreType.DMA,
    ],
    compiler_params=_sc_params,
)
def _spmm_kernel(hs_hbm, src_hbm, dst_hbm, cnt_hbm, out_hbm,
                 src_v, dst_v, cnt_v, rows0, rows1, acc, g0, g1, s0, s1):
    c = lax.axis_index("c")
    s = lax.axis_index("s")
    pltpu.sync_copy(cnt_hbm.at[c, s], cnt_v)
    cnt = cnt_v[...][0]

    def _zero(r, carry):
        for t in range(2):
            for j in range(F // 32):
                rows0[r, t, pl.ds(j * 16, 16)] = jnp.zeros((16,), jnp.float32)
        return carry

    lax.fori_loop(0, CH, _zero, 0)
    for k in range(SPT // CH):
        pltpu.sync_copy(rows0, acc.at[pl.ds(s * SPT + k * CH, CH)])
    plsc.subcore_barrier()

    rows = (rows0, rows1)
    gsem = (g0, g1)
    ssem = (s0, s1)

    def _gather_start(j, b):
        for u in range(4):
            pltpu.make_async_copy(
                hs_hbm.at[src_v.at[pl.ds(j * CH + u * 16, 16)]],
                rows[b].at[pl.ds(u * 16, 16)], gsem[b]).start()

    def _gather_wait(b):
        pltpu.make_async_copy(hs_hbm.at[src_v.at[pl.ds(0, CH)]],
                              rows[b], gsem[b]).wait()

    def _scatter(j, b):
        return pltpu.make_async_copy(rows[b], acc.at[dst_v.at[j]], ssem[b])

    nc0 = jnp.minimum((cnt + CH - 1) // CH, HCH)
    nc1 = (jnp.maximum(cnt - HLF, 0) + CH - 1) // CH
    for h, nc in ((0, nc0), (1, nc1)):
        pltpu.sync_copy(src_hbm.at[c, s, pl.ds(h * HLF, HLF)], src_v)
        pltpu.sync_copy(dst_hbm.at[c, s, h], dst_v)

        npair = jnp.maximum((nc + 1) // 2, 1)
        n2 = 2 * npair
        _gather_start(0, 0)
        _gather_start(1, 1)

        def _body(i, carry):
            j = 2 * i
            _gather_wait(0)
            _scatter(j, 0).start(add=True)
            _gather_wait(1)
            _scatter(j + 1, 1).start(add=True)
            _scatter(j, 0).wait()

            @pl.when(j + 2 < n2)
            def _():
                _gather_start(j + 2, 0)

            _scatter(j + 1, 1).wait()

            @pl.when(j + 3 < n2)
            def _():
                _gather_start(j + 3, 1)
            return carry

        lax.fori_loop(0, npair, _body, 0)
    plsc.subcore_barrier()
    sl = pl.ds(s * SPT, SPT)
    pltpu.sync_copy(acc.at[sl], out_hbm.at[c].at[sl])


def _prologue_body(x_ref, do_ref, di_ref, xs_ref, s_ref, d_ref):
    dgo = do_ref[...]
    dgi = di_ref[...]
    sv = jnp.where(dgo > 0, lax.rsqrt(jnp.maximum(dgo, 1.0)), 0.0)
    dv = jnp.where(dgi > 0, lax.rsqrt(jnp.maximum(dgi, 1.0)), 0.0)
    row = lax.broadcasted_iota(jnp.int32, (RB, 1), 0) + pl.program_id(0) * RB
    xs_ref[...] = jnp.where(row < N, x_ref[...] * sv, 0.0)
    s_ref[...] = sv
    d_ref[...] = dv


def _prologue(x, deg_out, deg_in):
    return pl.pallas_call(
        _prologue_body,
        grid=(GRID,),
        in_specs=[
            pl.BlockSpec((RB, F), lambda i: (i, 0)),
            pl.BlockSpec((RB, 1), lambda i: (i, 0)),
            pl.BlockSpec((RB, 1), lambda i: (i, 0)),
        ],
        out_specs=[
            pl.BlockSpec((RB, F), lambda i: (i, 0)),
            pl.BlockSpec((RB, 1), lambda i: (i, 0)),
            pl.BlockSpec((RB, 1), lambda i: (i, 0)),
        ],
        out_shape=[
            jax.ShapeDtypeStruct((NP, F), jnp.float32),
            jax.ShapeDtypeStruct((NP, 1), jnp.float32),
            jax.ShapeDtypeStruct((NP, 1), jnp.float32),
        ],
    )(x, deg_out, deg_in)


def _layer_body(m_ref, d_ref, s_ref, w_ref, b_ref, o_ref):
    y = jnp.dot(m_ref[...], w_ref[...],
                preferred_element_type=jnp.float32,
                precision=lax.Precision.HIGHEST)
    y = y * d_ref[...] + b_ref[...]
    o_ref[...] = jnp.tanh(y) * s_ref[...]


def _layer(m, dvec, svec, w, b):
    return pl.pallas_call(
        _layer_body,
        grid=(GRID,),
        in_specs=[
            pl.BlockSpec((RB, F), lambda i: (i, 0)),
            pl.BlockSpec((RB, 1), lambda i: (i, 0)),
            pl.BlockSpec((RB, 1), lambda i: (i, 0)),
            pl.BlockSpec((F, F), lambda i: (0, 0)),
            pl.BlockSpec((1, F), lambda i: (0, 0)),
        ],
        out_specs=pl.BlockSpec((RB, F), lambda i: (i, 0)),
        out_shape=jax.ShapeDtypeStruct((NP, F), jnp.float32),
    )(m, dvec, svec, w, b)


def _final_body(m_ref, d_ref, w_ref, b_ref, o_ref):
    y = jnp.dot(m_ref[...], w_ref[...],
                preferred_element_type=jnp.float32,
                precision=lax.Precision.HIGHEST)
    y = y * d_ref[...] + b_ref[...]
    y = y - jnp.max(y, axis=1, keepdims=True)
    ey = jnp.exp(y)
    o_ref[...] = ey / jnp.sum(ey, axis=1, keepdims=True)


def _final(m, dvec, w, b):
    return pl.pallas_call(
        _final_body,
        grid=(GRID,),
        in_specs=[
            pl.BlockSpec((RB, F), lambda i: (i, 0)),
            pl.BlockSpec((RB, 1), lambda i: (i, 0)),
            pl.BlockSpec((F, F), lambda i: (0, 0)),
            pl.BlockSpec((1, F), lambda i: (0, 0)),
        ],
        out_specs=pl.BlockSpec((RB, F), lambda i: (i, 0)),
        out_shape=jax.ShapeDtypeStruct((N, F), jnp.float32),
    )(m, dvec, w, b)


def kernel(graph, x, W1, b1, W2, b2, W3, b3):
    graph_r = graph.reshape(2, NS, EPT)

    deg = _deg_kernel(graph_r)
    deg_out = deg[0].reshape(NP, 1)
    deg_in = deg[1].reshape(NP, 1)
    esrc, edst, ecnt = _part_kernel(graph_r)

    xs, svec, dvec = _prologue(x, deg_out, deg_in)

    edst_r = edst.reshape(2, NS, 2, HCH, CH)

    def spmm(h):
        out = _spmm_kernel(h.reshape(NP, 2, F // 2), esrc, edst_r, ecnt)
        return out.reshape(NP, F)

    h1 = _layer(spmm(xs), dvec, svec, W1, b1.reshape(1, F))
    h2 = _layer(spmm(h1), dvec, svec, W2, b2.reshape(1, F))
    return _final(spmm(h2), dvec, W3, b3.reshape(1, F))
